# Initial kernel scaffold; baseline (speedup 1.0000x reference)
import jax
import jax.numpy as jnp
from jax.experimental import pallas as pl


def kernel(z, codebook):
    raise NotImplementedError("write your pallas kernel here")


# speedup vs baseline: 1.7222x; 1.7222x over previous
import functools

import jax
import jax.numpy as jnp
from jax import lax
from jax.experimental import pallas as pl
from jax.experimental.pallas import tpu as pltpu

_NUM_CODES = 512
_LATENT_DIM = 256
_BT = 1024


def _vq_tc_kernel(z_ref, cb_ref, idx_ref, minsum_ref):
    i = pl.program_id(0)
    zb = z_ref[...]
    cb = cb_ref[...]
    s = jnp.dot(zb, cb.T, preferred_element_type=jnp.float32)
    zz = jnp.sum(zb * zb, axis=1, keepdims=True)
    ee = jnp.sum(cb * cb, axis=1)[None, :]
    d = (zz - 2.0 * s) + ee
    dmin = jnp.min(d, axis=1, keepdims=True)
    iota = lax.broadcasted_iota(jnp.int32, d.shape, 1)
    idx = jnp.min(jnp.where(d == dmin, iota, _NUM_CODES), axis=1)
    idx_ref[0, 0, :] = idx

    @pl.when(i == 0)
    def _():
        minsum_ref[0, 0] = 0.0

    minsum_ref[0, 0] += jnp.sum(dmin)


def _vq_tc(z_flat, codebook):
    n_tok = z_flat.shape[0]
    grid = n_tok // _BT
    idx3, minsum = pl.pallas_call(
        _vq_tc_kernel,
        grid=(grid,),
        in_specs=[
            pl.BlockSpec((_BT, _LATENT_DIM), lambda i: (i, 0)),
            pl.BlockSpec((_NUM_CODES, _LATENT_DIM), lambda i: (0, 0)),
        ],
        out_specs=[
            pl.BlockSpec((1, 1, _BT), lambda i: (i, 0, 0)),
            pl.BlockSpec(memory_space=pltpu.SMEM),
        ],
        out_shape=[
            jax.ShapeDtypeStruct((grid, 1, _BT), jnp.int32),
            jax.ShapeDtypeStruct((1, 1), jnp.float32),
        ],
    )(z_flat, codebook)
    return idx3.reshape(n_tok), minsum[0, 0]


def _gather_tc_kernel(cb_ref, idx_ref, out_ref):
    idx = idx_ref[0, 0, :]
    oh = jnp.where(
        lax.broadcasted_iota(jnp.int32, (_BT, _NUM_CODES), 1) == idx[:, None],
        1.0,
        0.0,
    )
    out_ref[...] = jnp.dot(
        oh, cb_ref[...],
        preferred_element_type=jnp.float32,
        precision=lax.Precision.HIGHEST,
    )


def _gather_tc(codebook, idx):
    n_tok = idx.shape[0]
    grid = n_tok // _BT
    out = pl.pallas_call(
        _gather_tc_kernel,
        grid=(grid,),
        in_specs=[
            pl.BlockSpec((_NUM_CODES, _LATENT_DIM), lambda i: (0, 0)),
            pl.BlockSpec((1, 1, _BT), lambda i: (i, 0, 0)),
        ],
        out_specs=pl.BlockSpec((_BT, _LATENT_DIM), lambda i: (i, 0)),
        out_shape=jax.ShapeDtypeStruct((n_tok, _LATENT_DIM), jnp.float32),
    )(codebook, idx.reshape(grid, 1, _BT))
    return out


def kernel(z, codebook):
    B, C, H, W = z.shape
    z_flat = jnp.transpose(z, (0, 2, 3, 1)).reshape(-1, C)
    idx, minsum = _vq_tc(z_flat, codebook)
    z_q_flat = _gather_tc(codebook, idx)
    z_q = jnp.transpose(z_q_flat.reshape(B, H, W, C), (0, 3, 1, 2))
    loss = minsum * (1.25 / (B * C * H * W))
    return z_q, idx, loss


# speedup vs baseline: 2.0766x; 1.2058x over previous
import functools

import jax
import jax.numpy as jnp
from jax import lax
from jax.experimental import pallas as pl
from jax.experimental.pallas import tpu as pltpu

_NUM_CODES = 512
_LATENT_DIM = 256
_BT = 1024


def _vq_tc_kernel(z_ref, cb_ref, zq_ref, idx_ref, minsum_ref):
    i = pl.program_id(0)
    zb = z_ref[...]
    cb = cb_ref[...]
    s = jnp.dot(zb, cb.T, preferred_element_type=jnp.float32)
    zz = jnp.sum(zb * zb, axis=1, keepdims=True)
    ee = jnp.sum(cb * cb, axis=1)[None, :]
    d = (zz - 2.0 * s) + ee
    dmin = jnp.min(d, axis=1, keepdims=True)
    iota = lax.broadcasted_iota(jnp.int32, d.shape, 1)
    idx = jnp.min(jnp.where(d == dmin, iota, _NUM_CODES), axis=1)
    idx_ref[0, 0, :] = idx

    oh = jnp.where(
        lax.broadcasted_iota(jnp.int32, (_BT, _NUM_CODES), 1) == idx[:, None],
        1.0,
        0.0,
    )
    zq_ref[0, :, :] = lax.dot_general(
        cb, oh, (((0,), (1,)), ((), ())),
        preferred_element_type=jnp.float32,
    )

    @pl.when(i == 0)
    def _():
        minsum_ref[0, 0] = 0.0

    minsum_ref[0, 0] += jnp.sum(dmin)


def kernel(z, codebook):
    B, C, H, W = z.shape
    z_flat = jnp.transpose(z, (0, 2, 3, 1)).reshape(-1, C)
    n_tok = z_flat.shape[0]
    grid = n_tok // _BT
    zq_t, idx3, minsum = pl.pallas_call(
        _vq_tc_kernel,
        grid=(grid,),
        in_specs=[
            pl.BlockSpec((_BT, _LATENT_DIM), lambda i: (i, 0)),
            pl.BlockSpec((_NUM_CODES, _LATENT_DIM), lambda i: (0, 0)),
        ],
        out_specs=[
            pl.BlockSpec((1, _LATENT_DIM, _BT), lambda i: (i, 0, 0)),
            pl.BlockSpec((1, 1, _BT), lambda i: (i, 0, 0)),
            pl.BlockSpec(memory_space=pltpu.SMEM),
        ],
        out_shape=[
            jax.ShapeDtypeStruct((grid, _LATENT_DIM, _BT), jnp.float32),
            jax.ShapeDtypeStruct((grid, 1, _BT), jnp.int32),
            jax.ShapeDtypeStruct((1, 1), jnp.float32),
        ],
    )(z_flat, codebook)
    z_q = zq_t.reshape(B, C, H, W)
    loss = minsum[0, 0] * (1.25 / (B * C * H * W))
    return z_q, idx3.reshape(n_tok), loss
